# baseline (speedup 1.0000x reference)
import functools

import jax
import jax.numpy as jnp
from jax import lax
from jax.experimental import pallas as pl
from jax.experimental.pallas import tpu as pltpu
from jax.experimental.pallas import tpu_sc as plsc

N_FIELDS = 26
EMB_DIM = 49
PAD_DIM = 128
BATCH = 4096

_NC = 2
_NS = 16
_NW = _NC * _NS
_BPW = BATCH // _NW


@functools.partial(
    pl.kernel,
    mesh=plsc.VectorSubcoreMesh(core_axis_name="c", subcore_axis_name="s"),
    out_type=jax.ShapeDtypeStruct((N_FIELDS, BATCH, PAD_DIM), jnp.float32),
    scratch_types=[
        pltpu.VMEM((_BPW,), jnp.int32),
        pltpu.VMEM((_BPW,), jnp.int32),
        pltpu.VMEM((_BPW, PAD_DIM), jnp.float32),
        pltpu.VMEM((_BPW, PAD_DIM), jnp.float32),
        pltpu.SemaphoreType.DMA,
        pltpu.SemaphoreType.DMA,
    ],
)
def _embed_sc(*refs):
    feats = refs[:N_FIELDS]
    tables_all = refs[N_FIELDS]
    out = refs[N_FIELDS + 1]
    idx_a, idx_b, rows_a, rows_b, sem_a, sem_b = refs[N_FIELDS + 2:]

    wid = lax.axis_index("s") * _NC + lax.axis_index("c")
    base = wid * _BPW

    idxs = (idx_a, idx_b)
    bufs = (rows_a, rows_b)
    sems = (sem_a, sem_b)
    copies = []
    pltpu.sync_copy(feats[0].at[pl.ds(base, _BPW)], idxs[0])
    copies.append(
        pltpu.async_copy(tables_all.at[0].at[idxs[0]], bufs[0], sems[0]))
    for i in range(N_FIELDS):
        nxt = (i + 1) % 2
        if i + 1 < N_FIELDS:
            pltpu.sync_copy(feats[i + 1].at[pl.ds(base, _BPW)], idxs[nxt])
            copies.append(
                pltpu.async_copy(tables_all.at[i + 1].at[idxs[nxt]],
                                 bufs[nxt], sems[nxt])
            )
        copies[i].wait()
        pltpu.sync_copy(bufs[i % 2], out.at[i, pl.ds(base, _BPW), :])


def kernel(feat_00, feat_01, feat_02, feat_03, feat_04, feat_05, feat_06,
           feat_07, feat_08, feat_09, feat_10, feat_11, feat_12, feat_13,
           feat_14, feat_15, feat_16, feat_17, feat_18, feat_19, feat_20,
           feat_21, feat_22, feat_23, feat_24, feat_25,
           W_00, W_01, W_02, W_03, W_04, W_05, W_06, W_07, W_08, W_09,
           W_10, W_11, W_12, W_13, W_14, W_15, W_16, W_17, W_18, W_19,
           W_20, W_21, W_22, W_23, W_24, W_25):
    feats = (feat_00, feat_01, feat_02, feat_03, feat_04, feat_05, feat_06,
             feat_07, feat_08, feat_09, feat_10, feat_11, feat_12, feat_13,
             feat_14, feat_15, feat_16, feat_17, feat_18, feat_19, feat_20,
             feat_21, feat_22, feat_23, feat_24, feat_25)
    tables = (W_00, W_01, W_02, W_03, W_04, W_05, W_06, W_07, W_08, W_09,
              W_10, W_11, W_12, W_13, W_14, W_15, W_16, W_17, W_18, W_19,
              W_20, W_21, W_22, W_23, W_24, W_25)
    stacked = jnp.pad(jnp.stack(tables),
                      ((0, 0), (0, 0), (0, PAD_DIM - EMB_DIM)))
    out = _embed_sc(*feats, stacked)
    out = jnp.swapaxes(out, 0, 1)[:, :, :EMB_DIM]
    return out.reshape(BATCH, N_FIELDS * EMB_DIM)


# speedup vs baseline: 1.5645x; 1.5645x over previous
import functools

import jax
import jax.numpy as jnp
from jax import lax
from jax.experimental import pallas as pl
from jax.experimental.pallas import tpu as pltpu
from jax.experimental.pallas import tpu_sc as plsc

N_FIELDS = 26
EMB_DIM = 49
PAD_DIM = 128
BATCH = 4096

_NC = 2
_NS = 16
_NW = _NC * _NS
_BPW = BATCH // _NW


@functools.partial(
    pl.kernel,
    mesh=plsc.VectorSubcoreMesh(core_axis_name="c", subcore_axis_name="s"),
    out_type=jax.ShapeDtypeStruct((N_FIELDS, BATCH, PAD_DIM), jnp.float32),
    scratch_types=[
        pltpu.VMEM((_BPW,), jnp.int32),
        pltpu.VMEM((_BPW,), jnp.int32),
        pltpu.VMEM((_BPW, PAD_DIM), jnp.float32),
        pltpu.VMEM((_BPW, PAD_DIM), jnp.float32),
        pltpu.SemaphoreType.DMA,
        pltpu.SemaphoreType.DMA,
    ],
)
def _embed_sc(*refs):
    feats = refs[:N_FIELDS]
    tables = refs[N_FIELDS:2 * N_FIELDS]
    out = refs[2 * N_FIELDS]
    idx_a, idx_b, rows_a, rows_b, sem_a, sem_b = refs[2 * N_FIELDS + 1:]

    wid = lax.axis_index("s") * _NC + lax.axis_index("c")
    base = wid * _BPW

    idxs = (idx_a, idx_b)
    bufs = (rows_a, rows_b)
    sems = (sem_a, sem_b)
    copies = []
    pltpu.sync_copy(feats[0].at[pl.ds(base, _BPW)], idxs[0])
    copies.append(pltpu.async_copy(tables[0].at[idxs[0]], bufs[0], sems[0]))
    for i in range(N_FIELDS):
        nxt = (i + 1) % 2
        if i + 1 < N_FIELDS:
            pltpu.sync_copy(feats[i + 1].at[pl.ds(base, _BPW)], idxs[nxt])
            copies.append(
                pltpu.async_copy(tables[i + 1].at[idxs[nxt]], bufs[nxt],
                                 sems[nxt])
            )
        copies[i].wait()
        pltpu.sync_copy(bufs[i % 2], out.at[i, pl.ds(base, _BPW), :])


def kernel(feat_00, feat_01, feat_02, feat_03, feat_04, feat_05, feat_06,
           feat_07, feat_08, feat_09, feat_10, feat_11, feat_12, feat_13,
           feat_14, feat_15, feat_16, feat_17, feat_18, feat_19, feat_20,
           feat_21, feat_22, feat_23, feat_24, feat_25,
           W_00, W_01, W_02, W_03, W_04, W_05, W_06, W_07, W_08, W_09,
           W_10, W_11, W_12, W_13, W_14, W_15, W_16, W_17, W_18, W_19,
           W_20, W_21, W_22, W_23, W_24, W_25):
    feats = (feat_00, feat_01, feat_02, feat_03, feat_04, feat_05, feat_06,
             feat_07, feat_08, feat_09, feat_10, feat_11, feat_12, feat_13,
             feat_14, feat_15, feat_16, feat_17, feat_18, feat_19, feat_20,
             feat_21, feat_22, feat_23, feat_24, feat_25)
    tables = (W_00, W_01, W_02, W_03, W_04, W_05, W_06, W_07, W_08, W_09,
              W_10, W_11, W_12, W_13, W_14, W_15, W_16, W_17, W_18, W_19,
              W_20, W_21, W_22, W_23, W_24, W_25)
    padded = tuple(
        jnp.pad(W, ((0, 0), (0, PAD_DIM - EMB_DIM))) for W in tables
    )
    out = _embed_sc(*feats, *padded)
    out = jnp.swapaxes(out, 0, 1)[:, :, :EMB_DIM]
    return out.reshape(BATCH, N_FIELDS * EMB_DIM)
